# baseline (device time: 194749 ns/iter reference)
import os as _os

import jax
import jax.numpy as jnp
from jax import lax
from jax.experimental import pallas as pl
from jax.experimental.pallas import tpu as pltpu

N_DEV = 4
M = 4096
N = 2048
MC = M // N_DEV
NH = N // 2
N_HOP = 2 * (N_DEV - 1)
S = 4
MS = MC // S
N_SLOT = N_DEV - 1

_SKIP_DOTS = _os.environ.get("KERNEL_SKIP_DOTS") == "1"
_DIRS = (0,) if _os.environ.get("KERNEL_SINGLE_DIR") == "1" else (0, 1)


def _body(x_ref, w_ref, scale_ref, out_ref,
          send_buf, recv_buf, send_sems, recv_sems):
    my = lax.axis_index("i")
    left = (my + N_DEV - 1) % N_DEV
    right = (my + 1) % N_DEV
    nbr_to = (right, left)
    sc = scale_ref[0, 0]

    barrier = pltpu.get_barrier_semaphore()
    for nbr in (left, right):
        pl.semaphore_signal(barrier, inc=1, device_id=(nbr,),
                            device_id_type=pl.DeviceIdType.MESH)
    pl.semaphore_wait(barrier, 2)

    def subrows(c, j):
        return pl.ds(c * MC + j * MS, MS)

    def rows(c):
        return pl.ds(c * MC, MC)

    cols = (slice(0, NH), slice(NH, N))

    def dot_half(c, d):
        if _SKIP_DOTS:
            return
        out_ref[rows(c), cols[d]] = lax.dot_general(
            x_ref[rows(c), :], w_ref[:, cols[d]],
            (((1,), (0,)), ((), ())),
            preferred_element_type=jnp.float32,
        )

    def recv_chunk(h, d):
        if h < N_DEV - 1:
            return (my + N_DEV - 1 - h) % N_DEV if d == 0 else (my + 1 + h) % N_DEV
        s = h - (N_DEV - 1)
        return (my + 4 - s) % N_DEV if d == 0 else (my + 4 + s) % N_DEV

    def make_rdma(src, d, h, j):
        return pltpu.make_async_remote_copy(
            src_ref=src,
            dst_ref=recv_buf.at[d, h % N_SLOT, j],
            send_sem=send_sems.at[d, j],
            recv_sem=recv_sems.at[d, h % N_SLOT, j],
            device_id=(nbr_to[d],),
            device_id_type=pl.DeviceIdType.MESH,
        )

    dot_half(my, 0)

    prev = [[None] * S for _ in range(2)]
    deferred = []

    for h in range(N_HOP):
        slot = (h - 1) % N_SLOT
        for j in range(S):
            for d in _DIRS:
                if h > 0:
                    prev[d][j].wait_recv()
                    prev[d][j].wait_send()
                    r = recv_chunk(h - 1, d)

                if h == 0:
                    if d == 1 and j == 0:
                        dot_half(my, 1)
                    send_buf[d, j, :, :] = out_ref[
                        subrows(my, j), cols[d]].astype(jnp.bfloat16)
                    rdma = make_rdma(send_buf.at[d, j], d, h, j)
                elif h < N_DEV - 1:
                    send_buf[d, j, :, :] = (
                        out_ref[subrows(r, j), cols[d]]
                        + recv_buf[d, slot, j, :, :].astype(jnp.float32)
                    ).astype(jnp.bfloat16)
                    rdma = make_rdma(send_buf.at[d, j], d, h, j)
                elif h == N_DEV - 1:
                    y = jnp.maximum(
                        (out_ref[subrows(r, j), cols[d]]
                         + recv_buf[d, slot, j, :, :].astype(jnp.float32))
                        * sc,
                        0.0)
                    send_buf[d, j, :, :] = y.astype(jnp.bfloat16)
                    deferred.append((r, d, j, y, None))
                    rdma = make_rdma(send_buf.at[d, j], d, h, j)
                else:
                    deferred.append((r, d, j, None, slot))
                    rdma = make_rdma(recv_buf.at[d, slot, j], d, h, j)

                rdma.start()
                prev[d][j] = rdma

            for r, d, j, y, sl in deferred:
                if y is None:
                    out_ref[subrows(r, j), cols[d]] = recv_buf[
                        d, sl, j, :, :].astype(jnp.float32)
                else:
                    out_ref[subrows(r, j), cols[d]] = y
            deferred = []

        if h == 0:
            dot_half((my + 3) % N_DEV, 0)
            dot_half((my + 1) % N_DEV, 1)
            dot_half((my + 2) % N_DEV, 0)
            dot_half((my + 2) % N_DEV, 1)
            dot_half((my + 1) % N_DEV, 0)
            dot_half((my + 3) % N_DEV, 1)

    slot = (N_HOP - 1) % N_SLOT
    for j in range(S):
        for d in _DIRS:
            prev[d][j].wait_recv()
            r = recv_chunk(N_HOP - 1, d)
            out_ref[subrows(r, j), cols[d]] = recv_buf[
                d, slot, j, :, :].astype(jnp.float32)
            prev[d][j].wait_send()


def kernel(x, w_mat, scale_x, scale_w):
    xq = x.astype(jnp.float8_e4m3fn)
    wq = w_mat.astype(jnp.float8_e4m3fn)
    sc = (scale_x.astype(jnp.float32)
          * scale_w.astype(jnp.float32)).reshape(1, 1)
    return pl.pallas_call(
        _body,
        out_shape=jax.ShapeDtypeStruct((M, N), jnp.float32),
        in_specs=[
            pl.BlockSpec(memory_space=pltpu.VMEM),
            pl.BlockSpec(memory_space=pltpu.VMEM),
            pl.BlockSpec(memory_space=pltpu.SMEM),
        ],
        out_specs=pl.BlockSpec(memory_space=pltpu.VMEM),
        scratch_shapes=[
            pltpu.VMEM((2, S, MS, NH), jnp.bfloat16),
            pltpu.VMEM((2, N_SLOT, S, MS, NH), jnp.bfloat16),
            pltpu.SemaphoreType.DMA((2, S)),
            pltpu.SemaphoreType.DMA((2, N_SLOT, S)),
        ],
        compiler_params=pltpu.CompilerParams(
            collective_id=0,
            vmem_limit_bytes=62 * 1024 * 1024,
        ),
    )(xq, wq, sc)


# device time: 183866 ns/iter; 1.0592x vs baseline; 1.0592x over previous
import os as _os

import jax
import jax.numpy as jnp
from jax import lax
from jax.experimental import pallas as pl
from jax.experimental.pallas import tpu as pltpu

N_DEV = 4
M = 4096
N = 2048
MC = M // N_DEV
NH = N // 2
N_HOP = 2 * (N_DEV - 1)
S = 4
MS = MC // S
N_SLOT = N_DEV - 1

_SKIP_DOTS = _os.environ.get("KERNEL_SKIP_DOTS") == "1"
_DIRS = (0,) if _os.environ.get("KERNEL_SINGLE_DIR") == "1" else (0, 1)


def _body(x_hbm, w_hbm, scale_ref, out_ref,
          send_buf, recv_buf, send_sems, recv_sems,
          xq, wq, stage, stage_sems):
    my = lax.axis_index("i")
    left = (my + N_DEV - 1) % N_DEV
    right = (my + 1) % N_DEV
    nbr_to = (right, left)
    sc = scale_ref[0, 0]

    barrier = pltpu.get_barrier_semaphore()
    for nbr in (left, right):
        pl.semaphore_signal(barrier, inc=1, device_id=(nbr,),
                            device_id_type=pl.DeviceIdType.MESH)
    pl.semaphore_wait(barrier, 2)

    def subrows(c, j):
        return pl.ds(c * MC + j * MS, MS)

    def rows(c):
        return pl.ds(c * MC, MC)

    cols = (slice(0, NH), slice(NH, N))

    def dot_half(c, d):
        if _SKIP_DOTS:
            return
        out_ref[rows(c), cols[d]] = lax.dot_general(
            xq[rows(c), :], wq[:, cols[d]],
            (((1,), (0,)), ((), ())),
            preferred_element_type=jnp.float32,
        )

    def recv_chunk(h, d):
        if h < N_DEV - 1:
            return (my + N_DEV - 1 - h) % N_DEV if d == 0 else (my + 1 + h) % N_DEV
        s = h - (N_DEV - 1)
        return (my + 4 - s) % N_DEV if d == 0 else (my + 4 + s) % N_DEV

    def make_rdma(src, d, h, j):
        return pltpu.make_async_remote_copy(
            src_ref=src,
            dst_ref=recv_buf.at[d, h % N_SLOT, j],
            send_sem=send_sems.at[d, j],
            recv_sem=recv_sems.at[d, h % N_SLOT, j],
            device_id=(nbr_to[d],),
            device_id_type=pl.DeviceIdType.MESH,
        )

    def stage_x_chunk(c):
        if _SKIP_DOTS:
            return
        cp = pltpu.make_async_copy(
            x_hbm.at[rows(c), :], stage, stage_sems.at[0])
        cp.start()
        cp.wait()
        xq[rows(c), :] = stage[:, :].astype(jnp.float8_e4m3fn)

    def stage_w_half(d):
        if _SKIP_DOTS:
            return
        cp = pltpu.make_async_copy(
            w_hbm.at[:, cols[d]], stage, stage_sems.at[0])
        cp.start()
        cp.wait()
        wq[:, cols[d]] = stage[:, :].astype(jnp.float8_e4m3fn)

    stage_w_half(0)
    stage_x_chunk(my)
    dot_half(my, 0)

    prev = [[None] * S for _ in range(2)]
    deferred = []

    for h in range(N_HOP):
        slot = (h - 1) % N_SLOT
        for j in range(S):
            for d in _DIRS:
                if h > 0:
                    prev[d][j].wait_recv()
                    prev[d][j].wait_send()
                    r = recv_chunk(h - 1, d)

                if h == 0:
                    if d == 1 and j == 0:
                        stage_w_half(1)
                        dot_half(my, 1)
                    send_buf[d, j, :, :] = out_ref[
                        subrows(my, j), cols[d]].astype(jnp.bfloat16)
                    rdma = make_rdma(send_buf.at[d, j], d, h, j)
                elif h < N_DEV - 1:
                    send_buf[d, j, :, :] = (
                        out_ref[subrows(r, j), cols[d]]
                        + recv_buf[d, slot, j, :, :].astype(jnp.float32)
                    ).astype(jnp.bfloat16)
                    rdma = make_rdma(send_buf.at[d, j], d, h, j)
                elif h == N_DEV - 1:
                    y = jnp.maximum(
                        (out_ref[subrows(r, j), cols[d]]
                         + recv_buf[d, slot, j, :, :].astype(jnp.float32))
                        * sc,
                        0.0)
                    send_buf[d, j, :, :] = y.astype(jnp.bfloat16)
                    deferred.append((r, d, j, y, None))
                    rdma = make_rdma(send_buf.at[d, j], d, h, j)
                else:
                    deferred.append((r, d, j, None, slot))
                    rdma = make_rdma(recv_buf.at[d, slot, j], d, h, j)

                rdma.start()
                prev[d][j] = rdma

            for r, d, j, y, sl in deferred:
                if y is None:
                    out_ref[subrows(r, j), cols[d]] = recv_buf[
                        d, sl, j, :, :].astype(jnp.float32)
                else:
                    out_ref[subrows(r, j), cols[d]] = y
            deferred = []

        if h == 0:
            stage_x_chunk((my + 3) % N_DEV)
            dot_half((my + 3) % N_DEV, 0)
            stage_x_chunk((my + 1) % N_DEV)
            dot_half((my + 1) % N_DEV, 1)
            stage_x_chunk((my + 2) % N_DEV)
            dot_half((my + 2) % N_DEV, 0)
            dot_half((my + 2) % N_DEV, 1)
            dot_half((my + 1) % N_DEV, 0)
            dot_half((my + 3) % N_DEV, 1)

    slot = (N_HOP - 1) % N_SLOT
    for j in range(S):
        for d in _DIRS:
            prev[d][j].wait_recv()
            r = recv_chunk(N_HOP - 1, d)
            out_ref[subrows(r, j), cols[d]] = recv_buf[
                d, slot, j, :, :].astype(jnp.float32)
            prev[d][j].wait_send()


def kernel(x, w_mat, scale_x, scale_w):
    sc = (scale_x.astype(jnp.float32)
          * scale_w.astype(jnp.float32)).reshape(1, 1)
    return pl.pallas_call(
        _body,
        out_shape=jax.ShapeDtypeStruct((M, N), jnp.float32),
        in_specs=[
            pl.BlockSpec(memory_space=pl.ANY),
            pl.BlockSpec(memory_space=pl.ANY),
            pl.BlockSpec(memory_space=pltpu.SMEM),
        ],
        out_specs=pl.BlockSpec(memory_space=pltpu.VMEM),
        scratch_shapes=[
            pltpu.VMEM((2, S, MS, NH), jnp.bfloat16),
            pltpu.VMEM((2, N_SLOT, S, MS, NH), jnp.bfloat16),
            pltpu.SemaphoreType.DMA((2, S)),
            pltpu.SemaphoreType.DMA((2, N_SLOT, S)),
            pltpu.VMEM((M, 1024), jnp.float8_e4m3fn),
            pltpu.VMEM((1024, N), jnp.float8_e4m3fn),
            pltpu.VMEM((1024, 1024), jnp.float32),
            pltpu.SemaphoreType.DMA((1,)),
        ],
        compiler_params=pltpu.CompilerParams(
            collective_id=0,
            vmem_limit_bytes=62 * 1024 * 1024,
        ),
    )(x, w_mat, sc)


# device time: 182220 ns/iter; 1.0688x vs baseline; 1.0090x over previous
import os as _os

import jax
import jax.numpy as jnp
from jax import lax
from jax.experimental import pallas as pl
from jax.experimental.pallas import tpu as pltpu

N_DEV = 4
M = 4096
N = 2048
MC = M // N_DEV
NH = N // 2
N_HOP = 2 * (N_DEV - 1)
S = 4
MS = MC // S
N_SLOT = N_DEV - 1

_SKIP_DOTS = _os.environ.get("KERNEL_SKIP_DOTS") == "1"
_DIRS = (0,) if _os.environ.get("KERNEL_SINGLE_DIR") == "1" else (0, 1)


def _body(x_hbm, w_hbm, scale_ref, out_ref,
          send_buf, recv_buf, send_sems, recv_sems,
          xq, wq, stage, stage_sems):
    my = lax.axis_index("i")
    left = (my + N_DEV - 1) % N_DEV
    right = (my + 1) % N_DEV
    nbr_to = (right, left)
    sc = scale_ref[0, 0]

    barrier = pltpu.get_barrier_semaphore()
    for nbr in (left, right):
        pl.semaphore_signal(barrier, inc=1, device_id=(nbr,),
                            device_id_type=pl.DeviceIdType.MESH)
    pl.semaphore_wait(barrier, 2)

    def subrows(c, j):
        return pl.ds(c * MC + j * MS, MS)

    def rows(c):
        return pl.ds(c * MC, MC)

    cols = (slice(0, NH), slice(NH, N))

    def dot_half(c, d):
        if _SKIP_DOTS:
            return
        out_ref[rows(c), cols[d]] = lax.dot_general(
            xq[rows(c), :], wq[:, cols[d]],
            (((1,), (0,)), ((), ())),
            preferred_element_type=jnp.float32,
        )

    def recv_chunk(h, d):
        if h < N_DEV - 1:
            return (my + N_DEV - 1 - h) % N_DEV if d == 0 else (my + 1 + h) % N_DEV
        s = h - (N_DEV - 1)
        return (my + 4 - s) % N_DEV if d == 0 else (my + 4 + s) % N_DEV

    def make_rdma(src, d, h, j):
        return pltpu.make_async_remote_copy(
            src_ref=src,
            dst_ref=recv_buf.at[d, h % N_SLOT, j],
            send_sem=send_sems.at[d, j],
            recv_sem=recv_sems.at[d, h % N_SLOT, j],
            device_id=(nbr_to[d],),
            device_id_type=pl.DeviceIdType.MESH,
        )

    def stage_x_chunk(c, b=0):
        if _SKIP_DOTS:
            return
        cp = pltpu.make_async_copy(
            x_hbm.at[rows(c), :], stage.at[b], stage_sems.at[b])
        cp.start()
        cp.wait()
        xq[rows(c), :] = stage[b, :, :].astype(jnp.float8_e4m3fn)

    def stage_w_half(d, b=0):
        if _SKIP_DOTS:
            return
        cp = pltpu.make_async_copy(
            w_hbm.at[:, cols[d]], stage.at[b], stage_sems.at[b])
        cp.start()
        cp.wait()
        wq[:, cols[d]] = stage[b, :, :].astype(jnp.float8_e4m3fn)

    if not _SKIP_DOTS:
        cpw = pltpu.make_async_copy(
            w_hbm.at[:, cols[0]], stage.at[0], stage_sems.at[0])
        cpw.start()
        cpx = pltpu.make_async_copy(
            x_hbm.at[rows(my), :], stage.at[1], stage_sems.at[1])
        cpx.start()
        cpw.wait()
        wq[:, cols[0]] = stage[0, :, :].astype(jnp.float8_e4m3fn)
        cpx.wait()
        xq[rows(my), :] = stage[1, :, :].astype(jnp.float8_e4m3fn)
    dot_half(my, 0)

    prev = [[None] * S for _ in range(2)]
    deferred = []

    for h in range(N_HOP):
        slot = (h - 1) % N_SLOT
        for j in range(S):
            for d in _DIRS:
                if h > 0:
                    prev[d][j].wait_recv()
                    prev[d][j].wait_send()
                    r = recv_chunk(h - 1, d)

                if h == 0:
                    if d == 1 and j == 0:
                        stage_w_half(1, 1)
                        dot_half(my, 1)
                    send_buf[d, j, :, :] = out_ref[
                        subrows(my, j), cols[d]].astype(jnp.bfloat16)
                    rdma = make_rdma(send_buf.at[d, j], d, h, j)
                elif h < N_DEV - 1:
                    send_buf[d, j, :, :] = (
                        out_ref[subrows(r, j), cols[d]]
                        + recv_buf[d, slot, j, :, :].astype(jnp.float32)
                    ).astype(jnp.bfloat16)
                    rdma = make_rdma(send_buf.at[d, j], d, h, j)
                elif h == N_DEV - 1:
                    y = jnp.maximum(
                        (out_ref[subrows(r, j), cols[d]]
                         + recv_buf[d, slot, j, :, :].astype(jnp.float32))
                        * sc,
                        0.0)
                    send_buf[d, j, :, :] = y.astype(jnp.bfloat16)
                    deferred.append((r, d, j, y, None))
                    rdma = make_rdma(send_buf.at[d, j], d, h, j)
                else:
                    deferred.append((r, d, j, None, slot))
                    rdma = make_rdma(recv_buf.at[d, slot, j], d, h, j)

                rdma.start()
                prev[d][j] = rdma

            for r, d, j, y, sl in deferred:
                if y is None:
                    out_ref[subrows(r, j), cols[d]] = recv_buf[
                        d, sl, j, :, :].astype(jnp.float32)
                else:
                    out_ref[subrows(r, j), cols[d]] = y
            deferred = []

        if h == 0:
            stage_x_chunk((my + 3) % N_DEV, 0)
            dot_half((my + 3) % N_DEV, 0)
            stage_x_chunk((my + 1) % N_DEV, 1)
            dot_half((my + 1) % N_DEV, 1)
        elif h == 1:
            stage_x_chunk((my + 2) % N_DEV, 0)
            dot_half((my + 2) % N_DEV, 0)
            dot_half((my + 2) % N_DEV, 1)
        elif h == 2:
            dot_half((my + 1) % N_DEV, 0)
            dot_half((my + 3) % N_DEV, 1)

    slot = (N_HOP - 1) % N_SLOT
    for j in range(S):
        for d in _DIRS:
            prev[d][j].wait_recv()
            r = recv_chunk(N_HOP - 1, d)
            out_ref[subrows(r, j), cols[d]] = recv_buf[
                d, slot, j, :, :].astype(jnp.float32)
            prev[d][j].wait_send()


def kernel(x, w_mat, scale_x, scale_w):
    sc = (scale_x.astype(jnp.float32)
          * scale_w.astype(jnp.float32)).reshape(1, 1)
    return pl.pallas_call(
        _body,
        out_shape=jax.ShapeDtypeStruct((M, N), jnp.float32),
        in_specs=[
            pl.BlockSpec(memory_space=pl.ANY),
            pl.BlockSpec(memory_space=pl.ANY),
            pl.BlockSpec(memory_space=pltpu.SMEM),
        ],
        out_specs=pl.BlockSpec(memory_space=pltpu.VMEM),
        scratch_shapes=[
            pltpu.VMEM((2, S, MS, NH), jnp.bfloat16),
            pltpu.VMEM((2, N_SLOT, S, MS, NH), jnp.bfloat16),
            pltpu.SemaphoreType.DMA((2, S)),
            pltpu.SemaphoreType.DMA((2, N_SLOT, S)),
            pltpu.VMEM((M, 1024), jnp.float8_e4m3fn),
            pltpu.VMEM((1024, N), jnp.float8_e4m3fn),
            pltpu.VMEM((2, 1024, 1024), jnp.float32),
            pltpu.SemaphoreType.DMA((2,)),
        ],
        compiler_params=pltpu.CompilerParams(
            collective_id=0,
            vmem_limit_bytes=63 * 1024 * 1024,
        ),
    )(x, w_mat, sc)
